# Initial kernel scaffold; baseline (speedup 1.0000x reference)
import jax
import jax.numpy as jnp
from jax.experimental import pallas as pl


def kernel(x, w_t, b, gamma, beta):
    raise NotImplementedError("write your pallas kernel here")


# speedup vs baseline: 3.3868x; 3.3868x over previous
import functools

import jax
import jax.numpy as jnp
from jax.experimental import pallas as pl
from jax.experimental.pallas import tpu as pltpu

_EPS = 1e-5
_TAPS = {0: {0: 1}, 1: {0: 0, 1: 2}}


def _patches(x, hp, wp):
    m = hp * wp
    lane = jax.lax.broadcasted_iota(jnp.int32, (1, m), 1)
    last_col = (lane % wp) == (wp - 1)
    last_row = lane >= (hp - 1) * wp
    zero = jnp.zeros((), x.dtype)
    s01 = jnp.where(last_col, zero, jnp.roll(x, -1, axis=1))
    s10 = jnp.where(last_row, zero, jnp.roll(x, -wp, axis=1))
    s11 = jnp.where(last_col | last_row, zero, jnp.roll(x, -(wp + 1), axis=1))
    return jnp.concatenate([x, s01, s10, s11], axis=0)


def _conv_stats_kernel(x_ref, w_ref, psum_ref, psq_ref, *, hp, wp):
    p = _patches(x_ref[...], hp, wp)
    y = jnp.dot(w_ref[...], p, preferred_element_type=jnp.float32)
    psum_ref[...] = jnp.sum(y, axis=1, keepdims=True)
    psq_ref[...] = jnp.sum(y * y, axis=1, keepdims=True)


def _conv_bn_relu_kernel(x_ref, w_ref, scale_ref, shift_ref, o_ref, *, hp, wp):
    p = _patches(x_ref[...], hp, wp)
    y = jnp.dot(w_ref[...], p, preferred_element_type=jnp.float32)
    o_ref[...] = jnp.maximum(y * scale_ref[...] + shift_ref[...], 0.0)


def _phase_weights(w_t):
    cin, cout = w_t.shape[0], w_t.shape[1]
    wc = jnp.flip(w_t, axis=(2, 3)).transpose(1, 0, 2, 3).astype(jnp.float32)
    blocks = []
    for py in (0, 1):
        for px in (0, 1):
            cols = []
            for dy in (0, 1):
                for dx in (0, 1):
                    ky = _TAPS[py].get(dy)
                    kx = _TAPS[px].get(dx)
                    if ky is None or kx is None:
                        cols.append(jnp.zeros((cout, cin), jnp.float32))
                    else:
                        cols.append(wc[:, :, ky, kx])
            blocks.append(jnp.concatenate(cols, axis=1))
    return jnp.concatenate(blocks, axis=0)


@jax.jit
def _deconv_bn_relu(x, w_t, gamma, beta):
    n, cin, h, w = x.shape
    cout = w_t.shape[1]
    hp, wp = h, w
    m = hp * wp
    kdim, rdim = 4 * cin, 4 * cout

    w_stack = _phase_weights(w_t)
    xf = x.reshape(n, cin, m).astype(jnp.float32)

    cparams = pltpu.CompilerParams(
        dimension_semantics=("parallel",),
        vmem_limit_bytes=48 * 1024 * 1024)

    kern1 = functools.partial(_conv_stats_kernel, hp=hp, wp=wp)
    psum, psq = pl.pallas_call(
        kern1,
        out_shape=(jax.ShapeDtypeStruct((n, rdim, 1), jnp.float32),
                   jax.ShapeDtypeStruct((n, rdim, 1), jnp.float32)),
        grid=(n,),
        in_specs=[pl.BlockSpec((pl.Squeezed(), cin, m), lambda i: (i, 0, 0)),
                  pl.BlockSpec((rdim, kdim), lambda i: (0, 0))],
        out_specs=(pl.BlockSpec((pl.Squeezed(), rdim, 1), lambda i: (i, 0, 0)),
                   pl.BlockSpec((pl.Squeezed(), rdim, 1), lambda i: (i, 0, 0))),
        compiler_params=cparams,
    )(xf, w_stack)

    ho, wo = 2 * hp, 2 * wp
    count = float(n * ho * wo)
    ps = psum.reshape(n, 4, cout).sum(axis=(0, 1))
    pq = psq.reshape(n, 4, cout).sum(axis=(0, 1))
    mean = ps / count
    var = jnp.maximum(pq / count - mean * mean, 0.0)
    inv_std = jax.lax.rsqrt(var + _EPS)
    scale_c = gamma.astype(jnp.float32) * inv_std
    shift_c = beta.astype(jnp.float32) - mean * scale_c
    scale = jnp.tile(scale_c, 4).reshape(rdim, 1)
    shift = jnp.tile(shift_c, 4).reshape(rdim, 1)

    kern2 = functools.partial(_conv_bn_relu_kernel, hp=hp, wp=wp)
    o_all = pl.pallas_call(
        kern2,
        out_shape=jax.ShapeDtypeStruct((n, rdim, m), jnp.float32),
        grid=(n,),
        in_specs=[pl.BlockSpec((pl.Squeezed(), cin, m), lambda i: (i, 0, 0)),
                  pl.BlockSpec((rdim, kdim), lambda i: (0, 0)),
                  pl.BlockSpec((rdim, 1), lambda i: (0, 0)),
                  pl.BlockSpec((rdim, 1), lambda i: (0, 0))],
        out_specs=pl.BlockSpec((pl.Squeezed(), rdim, m), lambda i: (i, 0, 0)),
        compiler_params=cparams,
    )(xf, w_stack, scale, shift)

    out = o_all.reshape(n, 2, 2, cout, hp, wp)
    out = out.transpose(0, 3, 4, 1, 5, 2).reshape(n, cout, ho, wo)
    return out[:, :, 1:, 1:]


def kernel(x, w_t, b, gamma, beta):
    del b
    return _deconv_bn_relu(x, w_t, gamma, beta)


# speedup vs baseline: 3.7621x; 1.1108x over previous
import functools

import jax
import jax.numpy as jnp
from jax.experimental import pallas as pl
from jax.experimental.pallas import tpu as pltpu

_EPS = 1e-5
_TAPS = {0: {0: 1}, 1: {0: 0, 1: 2}}


def _patches(x, hp, wp):
    m = hp * wp
    lane = jax.lax.broadcasted_iota(jnp.int32, (1, m), 1)
    last_col = (lane % wp) == (wp - 1)
    last_row = lane >= (hp - 1) * wp
    zero = jnp.zeros((), x.dtype)
    s01 = jnp.where(last_col, zero, jnp.roll(x, -1, axis=1))
    s10 = jnp.where(last_row, zero, jnp.roll(x, -wp, axis=1))
    s11 = jnp.where(last_col | last_row, zero, jnp.roll(x, -(wp + 1), axis=1))
    return jnp.concatenate([x, s01, s10, s11], axis=0)


def _conv_stats_kernel(x_ref, w_ref, psum_ref, psq_ref, *, hp, wp):
    nb, cin, m = x_ref.shape
    rdim = psum_ref.shape[1]
    p = _patches(x_ref[...].reshape(nb * cin, m), hp, wp)
    y = jnp.dot(w_ref[...], p, preferred_element_type=jnp.float32)
    psum_ref[...] = jnp.sum(y, axis=1, keepdims=True).reshape(nb, rdim, 1)
    psq_ref[...] = jnp.sum(y * y, axis=1, keepdims=True).reshape(nb, rdim, 1)


def _conv_bn_relu_kernel(x_ref, w_ref, scale_ref, shift_ref, o_ref, *, hp, wp):
    nb, cin, m = x_ref.shape
    rdim = o_ref.shape[1]
    p = _patches(x_ref[...].reshape(nb * cin, m), hp, wp)
    y = jnp.dot(w_ref[...], p, preferred_element_type=jnp.float32)
    o = jnp.maximum(y * scale_ref[...] + shift_ref[...], 0.0)
    o_ref[...] = o.reshape(nb, rdim, m)


def _phase_weights(w_t):
    cin, cout = w_t.shape[0], w_t.shape[1]
    wc = jnp.flip(w_t, axis=(2, 3)).transpose(1, 0, 2, 3).astype(jnp.float32)
    blocks = []
    for py in (0, 1):
        for px in (0, 1):
            cols = []
            for dy in (0, 1):
                for dx in (0, 1):
                    ky = _TAPS[py].get(dy)
                    kx = _TAPS[px].get(dx)
                    if ky is None or kx is None:
                        cols.append(jnp.zeros((cout, cin), jnp.float32))
                    else:
                        cols.append(wc[:, :, ky, kx])
            blocks.append(jnp.concatenate(cols, axis=1))
    return jnp.concatenate(blocks, axis=0)


@jax.jit
def _deconv_bn_relu(x, w_t, gamma, beta):
    n, cin, h, w = x.shape
    cout = w_t.shape[1]
    hp, wp = h, w
    m = hp * wp
    kdim, rdim = 4 * cin, 4 * cout

    w_stack = _phase_weights(w_t)
    xf = x.reshape(n, cin, m).astype(jnp.float32)

    nb = 4
    ws3 = w_stack.reshape(rdim, 4, cin)
    w_big = jnp.zeros((nb, rdim, 4, nb, cin), jnp.float32)
    for bi in range(nb):
        w_big = w_big.at[bi, :, :, bi, :].set(ws3)
    w_big = w_big.reshape(nb * rdim, 4 * nb * cin)

    cparams = pltpu.CompilerParams(
        dimension_semantics=("parallel",),
        vmem_limit_bytes=48 * 1024 * 1024)

    kern1 = functools.partial(_conv_stats_kernel, hp=hp, wp=wp)
    psum, psq = pl.pallas_call(
        kern1,
        out_shape=(jax.ShapeDtypeStruct((n, rdim, 1), jnp.float32),
                   jax.ShapeDtypeStruct((n, rdim, 1), jnp.float32)),
        grid=(n // nb,),
        in_specs=[pl.BlockSpec((nb, cin, m), lambda i: (i, 0, 0)),
                  pl.BlockSpec((nb * rdim, nb * kdim), lambda i: (0, 0))],
        out_specs=(pl.BlockSpec((nb, rdim, 1), lambda i: (i, 0, 0)),
                   pl.BlockSpec((nb, rdim, 1), lambda i: (i, 0, 0))),
        compiler_params=cparams,
    )(xf, w_big)

    ho, wo = 2 * hp, 2 * wp
    count = float(n * ho * wo)
    ps = psum.reshape(n, 4, cout).sum(axis=(0, 1))
    pq = psq.reshape(n, 4, cout).sum(axis=(0, 1))
    mean = ps / count
    var = jnp.maximum(pq / count - mean * mean, 0.0)
    inv_std = jax.lax.rsqrt(var + _EPS)
    scale_c = gamma.astype(jnp.float32) * inv_std
    shift_c = beta.astype(jnp.float32) - mean * scale_c
    scale = jnp.tile(scale_c, 4 * nb).reshape(nb * rdim, 1)
    shift = jnp.tile(shift_c, 4 * nb).reshape(nb * rdim, 1)

    kern2 = functools.partial(_conv_bn_relu_kernel, hp=hp, wp=wp)
    o_all = pl.pallas_call(
        kern2,
        out_shape=jax.ShapeDtypeStruct((n, rdim, m), jnp.float32),
        grid=(n // nb,),
        in_specs=[pl.BlockSpec((nb, cin, m), lambda i: (i, 0, 0)),
                  pl.BlockSpec((nb * rdim, nb * kdim), lambda i: (0, 0)),
                  pl.BlockSpec((nb * rdim, 1), lambda i: (0, 0)),
                  pl.BlockSpec((nb * rdim, 1), lambda i: (0, 0))],
        out_specs=pl.BlockSpec((nb, rdim, m), lambda i: (i, 0, 0)),
        compiler_params=cparams,
    )(xf, w_big, scale, shift)

    out = o_all.reshape(n, 2, 2, cout, hp, wp)
    out = out.transpose(0, 3, 4, 1, 5, 2).reshape(n, cout, ho, wo)
    return out[:, :, 1:, 1:]


def kernel(x, w_t, b, gamma, beta):
    del b
    return _deconv_bn_relu(x, w_t, gamma, beta)
